# baseline (speedup 1.0000x reference)
import functools

import jax
import jax.numpy as jnp
from jax import lax
from jax.experimental import pallas as pl
from jax.experimental.pallas import tpu as pltpu
from jax.experimental.pallas import tpu_sc as plsc

VOCAB = 1000
D = 4
DPAD = 128
DK = 8
TPAD = 56


def _make_gather(n_tokens: int, chunk: int):
    info = plsc.get_sparse_core_info()
    nw = info.num_cores * info.num_subcores
    assert n_tokens % (nw * chunk) == 0
    b_per_w = n_tokens // nw
    n_chunks = b_per_w // chunk
    mesh = plsc.VectorSubcoreMesh(core_axis_name="c", subcore_axis_name="s")

    @functools.partial(
        pl.kernel,
        mesh=mesh,
        out_type=jax.ShapeDtypeStruct((n_tokens, DPAD), jnp.float32),
        scratch_types=[
            pltpu.VMEM((chunk,), jnp.int32),
            pltpu.VMEM((chunk, DPAD), jnp.float32),
            pltpu.SemaphoreType.DMA,
        ],
    )
    def gather_kernel(ids_hbm, emb_hbm, out_hbm, idx_v, rows_v, sem):
        wid = lax.axis_index("s") * info.num_cores + lax.axis_index("c")
        base = wid * b_per_w

        def body(i, carry):
            off = base + i * chunk
            pltpu.sync_copy(ids_hbm.at[pl.ds(off, chunk)], idx_v)
            pltpu.async_copy(emb_hbm.at[idx_v], rows_v, sem).wait()
            pltpu.sync_copy(rows_v, out_hbm.at[pl.ds(off, chunk)])
            return carry

        lax.fori_loop(0, n_chunks, body, 0)

    return gather_kernel


def _matmul_body(blk_b, h_ref, w_ref, out_ref):
    h8 = h_ref[:, :, :DK].reshape(blk_b * TPAD, DK)
    mm = lax.dot_general(
        h8, w_ref[...],
        dimension_numbers=(((1,), (1,)), ((), ())),
        preferred_element_type=jnp.float32,
    )
    for j in range(blk_b):
        out_ref[j] = mm[j * TPAD:j * TPAD + 50]


def _projection(h3, head8, b):
    blk_b = 8
    return pl.pallas_call(
        functools.partial(_matmul_body, blk_b),
        grid=(b // blk_b,),
        in_specs=[
            pl.BlockSpec((blk_b, TPAD, DPAD), lambda i: (i, 0, 0)),
            pl.BlockSpec((VOCAB, DK), lambda i: (0, 0)),
        ],
        out_specs=pl.BlockSpec((blk_b, 50, VOCAB), lambda i: (i, 0, 0)),
        out_shape=jax.ShapeDtypeStruct((b, 50, VOCAB), jnp.float32),
    )(h3, head8)


def kernel(input_ids, emb_weight, head_weight):
    b, t = input_ids.shape
    ids_pad = jnp.pad(input_ids.astype(jnp.int32), ((0, 0), (0, TPAD - t)))
    ids_flat = ids_pad.reshape(-1)
    emb_pad = jnp.pad(emb_weight, ((0, 0), (0, DPAD - D)))
    head8 = jnp.pad(head_weight, ((0, 0), (0, DK - D)))
    h2 = _make_gather(b * TPAD, 112)(ids_flat, emb_pad)
    h3 = h2.reshape(b, TPAD, DPAD)
    return _projection(h3, head8, b)


# speedup vs baseline: 1.6804x; 1.6804x over previous
import functools

import jax
import jax.numpy as jnp
from jax import lax
from jax.experimental import pallas as pl
from jax.experimental.pallas import tpu as pltpu
from jax.experimental.pallas import tpu_sc as plsc

VOCAB = 1000
D = 4
DPAD = 128
DK = 8
TPAD = 56
CHUNK = 112
NGRP = 4
NBUF = 2 * NGRP


def _make_gather(n_tokens: int):
    info = plsc.get_sparse_core_info()
    nw = info.num_cores * info.num_subcores
    assert n_tokens % (nw * CHUNK * 2 * NGRP) == 0
    b_per_w = n_tokens // nw
    n_grp2 = b_per_w // (CHUNK * 2 * NGRP)
    mesh = plsc.VectorSubcoreMesh(core_axis_name="c", subcore_axis_name="s")

    @functools.partial(
        pl.kernel,
        mesh=mesh,
        out_type=jax.ShapeDtypeStruct((n_tokens, DPAD), jnp.float32),
        scratch_types=(
            [pltpu.VMEM((CHUNK, DPAD), jnp.float32) for _ in range(NBUF)]
            + [pltpu.VMEM((CHUNK,), jnp.int32) for _ in range(NBUF)]
            + [pltpu.SemaphoreType.DMA] * 3
        ),
    )
    def gather_kernel(ids_hbm, emb_hbm, out_hbm, *refs):
        rows = refs[:NBUF]
        idx = refs[NBUF:2 * NBUF]
        i_sem, g_sem, w_sem = refs[2 * NBUF:]
        wid = lax.axis_index("s") * info.num_cores + lax.axis_index("c")
        base = wid * b_per_w

        def chunk_off(g, k):
            return base + (g * NGRP + k) * CHUNK

        def start_idx_loads(g, bufset):
            for k in range(NGRP):
                pltpu.async_copy(
                    ids_hbm.at[pl.ds(chunk_off(g, k), CHUNK)],
                    idx[bufset * NGRP + k], i_sem)

        def drain(sem, src, dst, n):
            for _ in range(n):
                pltpu.make_async_copy(src, dst, sem).wait()

        def run_group(j2, g, bufset):
            drain(i_sem, ids_hbm.at[pl.ds(0, CHUNK)], idx[0], NGRP)
            handles = []
            for k in range(NGRP):
                b = bufset * NGRP + k
                handles.append(pltpu.async_copy(
                    emb_hbm.at[idx[b]], rows[b], g_sem))

            for k in range(NGRP):
                handles[k].wait()

            @pl.when(j2 < n_grp2 - 1)
            def _():
                start_idx_loads(g + 2, bufset)
            for k in range(NGRP):
                b = bufset * NGRP + k
                pltpu.async_copy(
                    rows[b], out_hbm.at[pl.ds(chunk_off(g, k), CHUNK)],
                    w_sem)

        start_idx_loads(0, 0)
        start_idx_loads(1, 1)

        def body(j2, carry):
            @pl.when(j2 >= 1)
            def _():
                drain(w_sem, rows[0], out_hbm.at[pl.ds(0, CHUNK)], 2 * NGRP)

            run_group(j2, 2 * j2, 0)
            run_group(j2, 2 * j2 + 1, 1)
            return carry

        lax.fori_loop(0, n_grp2, body, 0)
        drain(w_sem, rows[0], out_hbm.at[pl.ds(0, CHUNK)], 2 * NGRP)

    return gather_kernel


def _matmul_body(h_ref, w_ref, out_ref):
    blk = h_ref.shape[1]
    h8 = h_ref[...].reshape(blk, DPAD)[:, :DK]
    mm = lax.dot_general(
        w_ref[...], h8,
        dimension_numbers=(((1,), (1,)), ((), ())),
        preferred_element_type=jnp.float32,
    )
    out_ref[...] = mm.reshape(1, VOCAB, blk)


def _projection(h3, head8, b, t):
    blk = 1024
    return pl.pallas_call(
        _matmul_body,
        grid=(t, b // blk),
        in_specs=[
            pl.BlockSpec((1, blk, DPAD), lambda ti, bi: (ti, bi, 0)),
            pl.BlockSpec((VOCAB, DK), lambda ti, bi: (0, 0)),
        ],
        out_specs=pl.BlockSpec((1, VOCAB, blk), lambda ti, bi: (ti, 0, bi)),
        out_shape=jax.ShapeDtypeStruct((t, VOCAB, b), jnp.float32),
    )(h3, head8)


def kernel(input_ids, emb_weight, head_weight):
    b, t = input_ids.shape
    ids_pad = jnp.pad(input_ids.astype(jnp.int32), ((0, 0), (0, TPAD - t)))
    ids_flat = ids_pad.T.reshape(-1)
    emb_pad = jnp.pad(emb_weight, ((0, 0), (0, DPAD - D)))
    head8 = jnp.pad(head_weight, ((0, 0), (0, DK - D)))
    h2 = _make_gather(b * TPAD)(ids_flat, emb_pad)
    h3 = h2.reshape(TPAD, b, DPAD)
    y = _projection(h3, head8, b, t)
    return jnp.transpose(y, (2, 0, 1))


# speedup vs baseline: 6.1894x; 3.6833x over previous
import functools

import jax
import jax.numpy as jnp
from jax import lax
from jax.experimental import pallas as pl
from jax.experimental.pallas import tpu as pltpu
from jax.experimental.pallas import tpu_sc as plsc

VOCAB = 1000
D = 4
DPAD = 128
DK = 8
TPAD = 56
CHUNK = 112
NGRP = 4
NBUF = 2 * NGRP


def _make_gather(n_tokens: int):
    info = plsc.get_sparse_core_info()
    nw = info.num_cores * info.num_subcores
    assert n_tokens % (nw * CHUNK * 2 * NGRP) == 0
    b_per_w = n_tokens // nw
    n_grp2 = b_per_w // (CHUNK * 2 * NGRP)
    mesh = plsc.VectorSubcoreMesh(core_axis_name="c", subcore_axis_name="s")

    @functools.partial(
        pl.kernel,
        mesh=mesh,
        out_type=jax.ShapeDtypeStruct((n_tokens, DPAD), jnp.float32),
        scratch_types=(
            [pltpu.VMEM((CHUNK, DPAD), jnp.float32) for _ in range(NBUF)]
            + [pltpu.VMEM((CHUNK,), jnp.int32) for _ in range(NBUF)]
            + [pltpu.VMEM_SHARED((VOCAB, DPAD), jnp.float32)]
            + [pltpu.SemaphoreType.DMA] * 3
        ),
    )
    def gather_kernel(ids_hbm, emb_hbm, out_hbm, *refs):
        rows = refs[:NBUF]
        idx = refs[NBUF:2 * NBUF]
        emb_sp = refs[2 * NBUF]
        i_sem, g_sem, w_sem = refs[2 * NBUF + 1:]
        wid = lax.axis_index("s") * info.num_cores + lax.axis_index("c")
        base = wid * b_per_w

        @pl.when(lax.axis_index("s") == 0)
        def _():
            pltpu.sync_copy(emb_hbm, emb_sp)

        plsc.subcore_barrier()

        def chunk_off(g, k):
            return base + (g * NGRP + k) * CHUNK

        def start_idx_loads(g, bufset):
            for k in range(NGRP):
                pltpu.async_copy(
                    ids_hbm.at[pl.ds(chunk_off(g, k), CHUNK)],
                    idx[bufset * NGRP + k], i_sem)

        def drain(sem, src, dst, n):
            for _ in range(n):
                pltpu.make_async_copy(src, dst, sem).wait()

        def run_group(j2, g, bufset):
            drain(i_sem, ids_hbm.at[pl.ds(0, CHUNK)], idx[0], NGRP)
            handles = []
            for k in range(NGRP):
                b = bufset * NGRP + k
                handles.append(pltpu.async_copy(
                    emb_sp.at[idx[b]], rows[b], g_sem))

            for k in range(NGRP):
                handles[k].wait()

            @pl.when(j2 < n_grp2 - 1)
            def _():
                start_idx_loads(g + 2, bufset)
            for k in range(NGRP):
                b = bufset * NGRP + k
                pltpu.async_copy(
                    rows[b], out_hbm.at[pl.ds(chunk_off(g, k), CHUNK)],
                    w_sem)

        start_idx_loads(0, 0)
        start_idx_loads(1, 1)

        def body(j2, carry):
            @pl.when(j2 >= 1)
            def _():
                drain(w_sem, rows[0], out_hbm.at[pl.ds(0, CHUNK)], 2 * NGRP)

            run_group(j2, 2 * j2, 0)
            run_group(j2, 2 * j2 + 1, 1)
            return carry

        lax.fori_loop(0, n_grp2, body, 0)
        drain(w_sem, rows[0], out_hbm.at[pl.ds(0, CHUNK)], 2 * NGRP)

    return gather_kernel


def _matmul_body(h_ref, w_ref, out_ref):
    blk = h_ref.shape[1]
    h8 = h_ref[...].reshape(blk, DPAD)[:, :DK]
    mm = lax.dot_general(
        w_ref[...], h8,
        dimension_numbers=(((1,), (1,)), ((), ())),
        preferred_element_type=jnp.float32,
    )
    out_ref[...] = mm.reshape(1, VOCAB, blk)


def _projection(h3, head8, b, t):
    blk = 1024
    return pl.pallas_call(
        _matmul_body,
        grid=(t, b // blk),
        in_specs=[
            pl.BlockSpec((1, blk, DPAD), lambda ti, bi: (ti, bi, 0)),
            pl.BlockSpec((VOCAB, DK), lambda ti, bi: (0, 0)),
        ],
        out_specs=pl.BlockSpec((1, VOCAB, blk), lambda ti, bi: (ti, 0, bi)),
        out_shape=jax.ShapeDtypeStruct((t, VOCAB, b), jnp.float32),
    )(h3, head8)


def kernel(input_ids, emb_weight, head_weight):
    b, t = input_ids.shape
    ids_pad = jnp.pad(input_ids.astype(jnp.int32), ((0, 0), (0, TPAD - t)))
    ids_flat = ids_pad.T.reshape(-1)
    emb_pad = jnp.pad(emb_weight, ((0, 0), (0, DPAD - D)))
    head8 = jnp.pad(head_weight, ((0, 0), (0, DK - D)))
    h2 = _make_gather(b * TPAD)(ids_flat, emb_pad)
    h3 = h2.reshape(TPAD, b, DPAD)
    y = _projection(h3, head8, b, t)
    return jnp.transpose(y, (2, 0, 1))


# speedup vs baseline: 7.6754x; 1.2401x over previous
import functools

import jax
import jax.numpy as jnp
from jax import lax
from jax.experimental import pallas as pl
from jax.experimental.pallas import tpu as pltpu
from jax.experimental.pallas import tpu_sc as plsc

VOCAB = 1000
D = 4
DK = 8
TPAD = 56
VEC = 16
WCHUNK = 896


def _make_gather(n_tokens: int):
    info = plsc.get_sparse_core_info()
    nw = info.num_cores * info.num_subcores
    assert n_tokens % (nw * 2 * WCHUNK) == 0
    b_per_w = n_tokens // nw
    n_pair = b_per_w // (2 * WCHUNK)
    nvec = WCHUNK // VEC
    mesh = plsc.VectorSubcoreMesh(core_axis_name="c", subcore_axis_name="s")

    @functools.partial(
        pl.kernel,
        mesh=mesh,
        out_type=jax.ShapeDtypeStruct((DK, n_tokens), jnp.float32),
        compiler_params=pltpu.CompilerParams(use_tc_tiling_on_sc=False, needs_layout_passes=False),
        scratch_types=(
            [pltpu.VMEM((VOCAB * DK,), jnp.float32)]
            + [pltpu.VMEM((b_per_w,), jnp.int32)]
            + [pltpu.VMEM((DK, WCHUNK), jnp.float32) for _ in range(2)]
            + [pltpu.SemaphoreType.DMA] * 2
        ),
    )
    def gather_kernel(ids_hbm, emb_hbm, out_hbm, emb_v, ids_v, wb0, wb1,
                      l_sem, w_sem):
        wid = lax.axis_index("s") * info.num_cores + lax.axis_index("c")
        base = wid * b_per_w
        pltpu.async_copy(emb_hbm, emb_v, l_sem)
        pltpu.async_copy(ids_hbm.at[pl.ds(base, b_per_w)], ids_v, l_sem)
        pltpu.make_async_copy(emb_hbm, emb_v, l_sem).wait()
        pltpu.make_async_copy(ids_hbm.at[pl.ds(0, b_per_w)], ids_v,
                              l_sem).wait()

        def fill(cc, wb):
            def vec_body(v, carry):
                ids16 = ids_v[pl.ds(cc * WCHUNK + v * VEC, VEC)]
                flat = ids16 * DK
                for d in range(DK):
                    vals = plsc.load_gather(emb_v, [flat + d])
                    wb[d, pl.ds(v * VEC, VEC)] = vals
                return carry

            lax.fori_loop(0, nvec, vec_body, 0)

        def flush(cc, wb):
            pltpu.async_copy(
                wb, out_hbm.at[:, pl.ds(base + cc * WCHUNK, WCHUNK)], w_sem)

        def wb_drain(n):
            for _ in range(n):
                pltpu.make_async_copy(
                    wb0, out_hbm.at[:, pl.ds(0, WCHUNK)], w_sem).wait()

        def body(p, carry):
            @pl.when(p >= 1)
            def _():
                wb_drain(2)

            fill(2 * p, wb0)
            flush(2 * p, wb0)
            fill(2 * p + 1, wb1)
            flush(2 * p + 1, wb1)
            return carry

        lax.fori_loop(0, n_pair, body, 0)
        wb_drain(2)

    return gather_kernel


def _matmul_body(h_ref, w_ref, out_ref):
    blk = h_ref.shape[1]
    mm = lax.dot_general(
        w_ref[...], h_ref[...],
        dimension_numbers=(((1,), (0,)), ((), ())),
        preferred_element_type=jnp.float32,
    )
    out_ref[...] = mm.reshape(1, VOCAB, blk)


def _projection(ht, head8, b, t):
    blk = 1024
    nblk = b // blk
    return pl.pallas_call(
        _matmul_body,
        grid=(t, nblk),
        in_specs=[
            pl.BlockSpec((DK, blk), lambda ti, bi: (0, ti * nblk + bi)),
            pl.BlockSpec((VOCAB, DK), lambda ti, bi: (0, 0)),
        ],
        out_specs=pl.BlockSpec((1, VOCAB, blk), lambda ti, bi: (ti, 0, bi)),
        out_shape=jax.ShapeDtypeStruct((t, VOCAB, b), jnp.float32),
    )(ht, head8)


def kernel(input_ids, emb_weight, head_weight):
    b, t = input_ids.shape
    ids_pad = jnp.pad(input_ids.astype(jnp.int32), ((0, 0), (0, TPAD - t)))
    ids_flat = ids_pad.T.reshape(-1)
    emb8 = jnp.pad(emb_weight, ((0, 0), (0, DK - D))).reshape(-1)
    head8 = jnp.pad(head_weight, ((0, 0), (0, DK - D)))
    ht = _make_gather(b * TPAD)(ids_flat, emb8)
    y = _projection(ht, head8, b, t)
    return jnp.transpose(y, (2, 0, 1))


# speedup vs baseline: 8.2984x; 1.0812x over previous
import functools

import jax
import jax.numpy as jnp
from jax import lax
from jax.experimental import pallas as pl
from jax.experimental.pallas import tpu as pltpu
from jax.experimental.pallas import tpu_sc as plsc

VOCAB = 1000
D = 4
DK = 8
TPAD = 56
VEC = 16
WCHUNK = 896


def _make_gather(n_tokens: int):
    info = plsc.get_sparse_core_info()
    nw = info.num_cores * info.num_subcores
    assert n_tokens % (nw * 2 * WCHUNK) == 0
    b_per_w = n_tokens // nw
    n_pair = b_per_w // (2 * WCHUNK)
    nvec = WCHUNK // VEC
    mesh = plsc.VectorSubcoreMesh(core_axis_name="c", subcore_axis_name="s")

    @functools.partial(
        pl.kernel,
        mesh=mesh,
        out_type=jax.ShapeDtypeStruct((DK, n_tokens), jnp.float32),
        compiler_params=pltpu.CompilerParams(use_tc_tiling_on_sc=False, needs_layout_passes=False),
        scratch_types=(
            [pltpu.VMEM((VOCAB * DK,), jnp.float32)]
            + [pltpu.VMEM((b_per_w,), jnp.int32)]
            + [pltpu.VMEM((DK, WCHUNK), jnp.float32) for _ in range(2)]
            + [pltpu.SemaphoreType.DMA] * 2
        ),
    )
    def gather_kernel(ids_hbm, emb_hbm, out_hbm, emb_v, ids_v, wb0, wb1,
                      l_sem, w_sem):
        wid = lax.axis_index("s") * info.num_cores + lax.axis_index("c")
        base = wid * b_per_w
        pltpu.async_copy(emb_hbm, emb_v, l_sem)
        pltpu.async_copy(ids_hbm.at[pl.ds(base, b_per_w)], ids_v, l_sem)
        pltpu.make_async_copy(emb_hbm, emb_v, l_sem).wait()
        pltpu.make_async_copy(ids_hbm.at[pl.ds(0, b_per_w)], ids_v,
                              l_sem).wait()

        def fill(cc, wb):
            def vec_body(v, carry):
                ids16 = ids_v[pl.ds(cc * WCHUNK + v * VEC, VEC)]
                flat = ids16 * DK
                for d in range(DK):
                    vals = plsc.load_gather(emb_v, [flat + d])
                    wb[d, pl.ds(v * VEC, VEC)] = vals
                return carry

            lax.fori_loop(0, nvec, vec_body, 0)

        def flush(cc, wb):
            pltpu.async_copy(
                wb, out_hbm.at[:, pl.ds(base + cc * WCHUNK, WCHUNK)], w_sem)

        def wb_drain(n):
            for _ in range(n):
                pltpu.make_async_copy(
                    wb0, out_hbm.at[:, pl.ds(0, WCHUNK)], w_sem).wait()

        def body(p, carry):
            @pl.when(p >= 1)
            def _():
                wb_drain(2)

            fill(2 * p, wb0)
            flush(2 * p, wb0)
            fill(2 * p + 1, wb1)
            flush(2 * p + 1, wb1)
            return carry

        lax.fori_loop(0, n_pair, body, 0)
        wb_drain(2)

    return gather_kernel


def _matmul_body(h_ref, w_ref, out_ref):
    blk = h_ref.shape[1]
    mm = lax.dot_general(
        w_ref[...], h_ref[...],
        dimension_numbers=(((1,), (0,)), ((), ())),
        preferred_element_type=jnp.float32,
    )
    out_ref[...] = mm.reshape(1, VOCAB, blk)


def _projection(ht, head8, b, t):
    blk = 2048
    nblk = b // blk
    return pl.pallas_call(
        _matmul_body,
        grid=(t, nblk),
        in_specs=[
            pl.BlockSpec((DK, blk), lambda ti, bi: (0, ti * nblk + bi)),
            pl.BlockSpec((VOCAB, DK), lambda ti, bi: (0, 0)),
        ],
        out_specs=pl.BlockSpec((1, VOCAB, blk), lambda ti, bi: (ti, 0, bi)),
        out_shape=jax.ShapeDtypeStruct((t, VOCAB, b), jnp.float32),
    )(ht, head8)


def kernel(input_ids, emb_weight, head_weight):
    b, t = input_ids.shape
    ids_pad = jnp.pad(input_ids.astype(jnp.int32), ((0, 0), (0, TPAD - t)))
    ids_flat = ids_pad.T.reshape(-1)
    emb8 = jnp.pad(emb_weight, ((0, 0), (0, DK - D))).reshape(-1)
    head8 = jnp.pad(head_weight, ((0, 0), (0, DK - D)))
    ht = _make_gather(b * TPAD)(ids_flat, emb8)
    y = _projection(ht, head8, b, t)
    return jnp.transpose(y, (2, 0, 1))
